# baseline (speedup 1.0000x reference)
import jax
import jax.numpy as jnp
from jax import lax
from jax.experimental import pallas as pl
from jax.experimental.pallas import tpu as pltpu
from jax.experimental.pallas import tpu_sc as plsc

B = 64
K = 100
KP = 128
H = 512.0
W = 512.0
NW = 32
ROWS = B // NW
NV = KP // 16

_DN = lax.GatherDimensionNumbers(offset_dims=(), collapsed_slice_dims=(0,),
                                 start_index_map=(0,))


def _gather16(v, idx):
    return lax.gather(v, idx[:, None], _DN, (1,),
                      mode=lax.GatherScatterMode.PROMISE_IN_BOUNDS)


def _allmax(v, lanes):
    for k in (1, 2, 4, 8):
        v = jnp.maximum(v, _gather16(v, lanes ^ k))
    return v


def _sc_body(boxes_hbm, valid_hbm, params_hbm, pv_out, misc_out,
             params_v, idx_v, boxes_v, valid_v, pv_v, misc_v,
             sem_b, sem_v):
    wid = lax.axis_index("s") * 2 + lax.axis_index("c")
    base = wid * ROWS
    lanes = lax.iota(jnp.int32, 16)

    pltpu.sync_copy(params_hbm.at[pl.ds(base, ROWS)], params_v)

    src_rows = []
    for i in range(ROWS):
        prow = params_v[i, :]
        r_b = jnp.full((16,), prow[0], jnp.float32)
        b_f = (base + i).astype(jnp.float32)
        src_rows.append(r_b + jnp.where(r_b >= b_f, 1.0, 0.0))
    idx_f = jnp.where(lanes == 1, src_rows[1], src_rows[0])
    idx_v[:] = idx_f.astype(jnp.int32)
    idx2 = idx_v.at[pl.ds(0, ROWS)]
    cp_b = pltpu.async_copy(boxes_hbm.at[idx2], boxes_v, sem_b)
    cp_v = pltpu.async_copy(valid_hbm.at[idx2], valid_v, sem_v)
    cp_b.wait()
    cp_v.wait()

    for i in range(ROWS):
        prow = params_v[i, :]
        scale_b = jnp.full((16,), prow[1], jnp.float32) * 1.5 + 0.5
        maxw = jnp.zeros((16,), jnp.float32)
        maxh = jnp.zeros((16,), jnp.float32)
        for j in range(NV):
            s = pl.ds(16 * j, 16)
            x1 = boxes_v[i, 0, s]
            y1 = boxes_v[i, 1, s]
            x2 = boxes_v[i, 2, s]
            y2 = boxes_v[i, 3, s]
            sw = (x2 - x1) * scale_b
            sh = (y2 - y1) * scale_b
            maxw = jnp.maximum(maxw, sw)
            maxh = jnp.maximum(maxh, sh)
            fits = (sh <= H) & (sw <= W)
            pv = fits & (valid_v[i, s] > 0.5)
            pv_v[i, s] = jnp.where(pv, 1, 0).astype(jnp.int32)
        mh = _allmax(maxh, lanes)
        mw = _allmax(maxw, lanes)
        ty = jnp.full((16,), prow[2], jnp.float32) * jnp.maximum(H - mh, 0.0)
        tx = jnp.full((16,), prow[3], jnp.float32) * jnp.maximum(W - mw, 0.0)
        u_fl = jnp.full((16,), prow[4], jnp.float32)
        flip = jnp.where(u_fl < 0.5, 1.0, 0.0)
        zero = jnp.zeros((16,), jnp.float32)
        misc = jnp.where(lanes == 0, ty,
               jnp.where(lanes == 1, tx,
               jnp.where(lanes == 2, scale_b,
               jnp.where(lanes == 3, src_rows[i],
               jnp.where(lanes == 4, flip, zero)))))
        misc_v[i, :] = misc

    pltpu.sync_copy(pv_v, pv_out.at[pl.ds(base, ROWS)])
    pltpu.sync_copy(misc_v, misc_out.at[pl.ds(base, ROWS)])


def kernel(images, boxes, instance_valid):
    del images

    key = jax.random.key(42)
    k_src, k_scale, k_ty, k_tx, k_flip = jax.random.split(key, 5)
    r = jax.random.randint(k_src, (B,), 0, B - 1)
    u_scale = jax.random.uniform(k_scale, (B,), dtype=jnp.float32)
    u_ty = jax.random.uniform(k_ty, (B,), dtype=jnp.float32)
    u_tx = jax.random.uniform(k_tx, (B,), dtype=jnp.float32)
    u_flip = jax.random.uniform(k_flip, (B,), dtype=jnp.float32)
    params = jnp.stack([r.astype(jnp.float32), u_scale, u_ty, u_tx, u_flip],
                       axis=-1)
    params = jnp.pad(params, ((0, 0), (0, 11)))

    boxes_t = jnp.pad(jnp.transpose(boxes, (0, 2, 1)),
                      ((0, 0), (0, 0), (0, KP - K)))
    valid_f = jnp.pad(instance_valid.astype(jnp.float32),
                      ((0, 0), (0, KP - K)))

    mesh = plsc.VectorSubcoreMesh(core_axis_name="c", subcore_axis_name="s")
    pv_i, misc = pl.kernel(
        _sc_body,
        out_type=(jax.ShapeDtypeStruct((B, KP), jnp.int32),
                  jax.ShapeDtypeStruct((B, 16), jnp.float32)),
        mesh=mesh,
        scratch_types=[
            pltpu.VMEM((ROWS, 16), jnp.float32),
            pltpu.VMEM((16,), jnp.int32),
            pltpu.VMEM((ROWS, 4, KP), jnp.float32),
            pltpu.VMEM((ROWS, KP), jnp.float32),
            pltpu.VMEM((ROWS, KP), jnp.int32),
            pltpu.VMEM((ROWS, 16), jnp.float32),
            pltpu.SemaphoreType.DMA,
            pltpu.SemaphoreType.DMA,
        ],
    )(boxes_t, valid_f, params)

    source_idx = misc[:, 3].astype(jnp.int64)
    translate = misc[:, 0:2]
    scale = misc[:, 2]
    hflip = misc[:, 4] > 0.5
    paste_valid = pv_i[:, :K] != 0
    return (source_idx, translate, scale, hflip, paste_valid)


# speedup vs baseline: 1.0479x; 1.0479x over previous
import jax
import jax.numpy as jnp
from jax import lax
from jax.experimental import pallas as pl
from jax.experimental.pallas import tpu as pltpu
from jax.experimental.pallas import tpu_sc as plsc

B = 64
K = 100
KP = 128
H = 512.0
W = 512.0
NW = 16
ROWS = B // NW
NV = KP // 16
MC = KP + 16

_DN = lax.GatherDimensionNumbers(offset_dims=(), collapsed_slice_dims=(0,),
                                 start_index_map=(0,))


def _gather16(v, idx):
    return lax.gather(v, idx[:, None], _DN, (1,),
                      mode=lax.GatherScatterMode.PROMISE_IN_BOUNDS)


def _allmax(v, lanes):
    for k in (1, 2, 4, 8):
        v = jnp.maximum(v, _gather16(v, lanes ^ k))
    return v


def _sc_body(comb_hbm, params_hbm, out_hbm, params_v, idx_v, comb_v, out_v, sem):
    wid = lax.axis_index("s")
    base = wid * ROWS
    lanes = lax.iota(jnp.int32, 16)

    pltpu.sync_copy(params_hbm.at[pl.ds(base, ROWS)], params_v)

    src_rows = []
    for i in range(ROWS):
        prow = params_v[i, :]
        r_b = jnp.full((16,), prow[0], jnp.float32)
        b_f = (base + i).astype(jnp.float32)
        src_rows.append(r_b + jnp.where(r_b >= b_f, 1.0, 0.0))
    idx_f = src_rows[0]
    for i in range(1, ROWS):
        idx_f = jnp.where(lanes == i, src_rows[i], idx_f)
    idx_v[:] = idx_f.astype(jnp.int32)
    pltpu.async_copy(comb_hbm.at[idx_v.at[pl.ds(0, ROWS)]], comb_v, sem).wait()

    for i in range(ROWS):
        prow = params_v[i, :]
        scale_b = jnp.full((16,), prow[1], jnp.float32) * 1.5 + 0.5
        maxw = jnp.zeros((16,), jnp.float32)
        maxh = jnp.zeros((16,), jnp.float32)
        for j in range(NV):
            s = pl.ds(16 * j, 16)
            x1 = comb_v[i, 0, s]
            y1 = comb_v[i, 1, s]
            x2 = comb_v[i, 2, s]
            y2 = comb_v[i, 3, s]
            sw = (x2 - x1) * scale_b
            sh = (y2 - y1) * scale_b
            maxw = jnp.maximum(maxw, sw)
            maxh = jnp.maximum(maxh, sh)
            fits = (sh <= H) & (sw <= W)
            pv = fits & (comb_v[i, 4, s] > 0.5)
            out_v[i, s] = jnp.where(pv, 1, 0).astype(jnp.int32)
        mh = _allmax(maxh, lanes)
        mw = _allmax(maxw, lanes)
        ty = jnp.full((16,), prow[2], jnp.float32) * jnp.maximum(H - mh, 0.0)
        tx = jnp.full((16,), prow[3], jnp.float32) * jnp.maximum(W - mw, 0.0)
        u_fl = jnp.full((16,), prow[4], jnp.float32)
        flip = jnp.where(u_fl < 0.5, 1.0, 0.0)
        zero = jnp.zeros((16,), jnp.float32)
        misc = jnp.where(lanes == 0, ty,
               jnp.where(lanes == 1, tx,
               jnp.where(lanes == 2, scale_b,
               jnp.where(lanes == 3, src_rows[i],
               jnp.where(lanes == 4, flip, zero)))))
        out_v[i, pl.ds(KP, 16)] = lax.bitcast_convert_type(misc, jnp.int32)

    pltpu.sync_copy(out_v, out_hbm.at[pl.ds(base, ROWS)])


def kernel(images, boxes, instance_valid):
    del images

    key = jax.random.key(42)
    k_src, k_scale, k_ty, k_tx, k_flip = jax.random.split(key, 5)
    r = jax.random.randint(k_src, (B,), 0, B - 1)
    u_scale = jax.random.uniform(k_scale, (B,), dtype=jnp.float32)
    u_ty = jax.random.uniform(k_ty, (B,), dtype=jnp.float32)
    u_tx = jax.random.uniform(k_tx, (B,), dtype=jnp.float32)
    u_flip = jax.random.uniform(k_flip, (B,), dtype=jnp.float32)
    params = jnp.stack([r.astype(jnp.float32), u_scale, u_ty, u_tx, u_flip],
                       axis=-1)
    params = jnp.pad(params, ((0, 0), (0, 11)))

    boxes_t = jnp.pad(jnp.transpose(boxes, (0, 2, 1)),
                      ((0, 0), (0, 0), (0, KP - K)))
    valid_f = jnp.pad(instance_valid.astype(jnp.float32),
                      ((0, 0), (0, KP - K)))
    comb = jnp.concatenate([boxes_t, valid_f[:, None, :]], axis=1)

    mesh = plsc.VectorSubcoreMesh(core_axis_name="c", subcore_axis_name="s",
                                  num_cores=1)
    out = pl.kernel(
        _sc_body,
        out_type=jax.ShapeDtypeStruct((B, MC), jnp.int32),
        mesh=mesh,
        scratch_types=[
            pltpu.VMEM((ROWS, 16), jnp.float32),
            pltpu.VMEM((16,), jnp.int32),
            pltpu.VMEM((ROWS, 5, KP), jnp.float32),
            pltpu.VMEM((ROWS, MC), jnp.int32),
            pltpu.SemaphoreType.DMA,
        ],
    )(comb, params)

    misc = lax.bitcast_convert_type(out[:, KP:KP + 16], jnp.float32)
    source_idx = misc[:, 3].astype(jnp.int64)
    translate = misc[:, 0:2]
    scale = misc[:, 2]
    hflip = misc[:, 4] > 0.5
    paste_valid = out[:, :K] != 0
    return (source_idx, translate, scale, hflip, paste_valid)
